# Initial kernel scaffold; baseline (speedup 1.0000x reference)
import jax
import jax.numpy as jnp
from jax.experimental import pallas as pl


def kernel(core0, core1, core2, x):
    raise NotImplementedError("write your pallas kernel here")


# speedup vs baseline: 1.4803x; 1.4803x over previous
import functools

import jax
import jax.numpy as jnp
from jax import lax
from jax.experimental import pallas as pl
from jax.experimental.pallas import tpu as pltpu
from jax.experimental.pallas import tpu_sc as plsc

V1, V2, V3 = 50, 40, 50
E1, E2, E3 = 4, 4, 4
R1, R2 = 16, 16
VOCAB = V1 * V2 * V3
EMB = E1 * E2 * E3
TROW = 2 * EMB
NV12 = V1 * V2
KDIM = E1 * E2 * R2
NCOLS = V3 * TROW
ROWBLK = 400
OROWS = ROWBLK * V3

NC, NS = 2, 16
NW = NC * NS
B_TOTAL = 4096 * 26
B_PER_W = B_TOTAL // NW
KCH = 128
NCH = B_PER_W // KCH


def _w12_body(c0_ref, c1_ref, out_ref):
    m0 = c0_ref[...].reshape(V1 * E1, R1)
    c1 = c1_ref[...].reshape(R1, V2 * E2 * R2)
    p = jnp.dot(m0, c1, preferred_element_type=jnp.float32)
    w12 = p.reshape(V1, E1, V2, E2 * R2).transpose(0, 2, 1, 3)
    out_ref[...] = w12.reshape(NV12, KDIM)


def _table_body(w12_ref, c2f_ref, out_ref):
    c2f = c2f_ref[...]
    pr = lax.broadcasted_iota(jnp.int32, (KDIM, R2), 0)
    rc = lax.broadcasted_iota(jnp.int32, (KDIM, R2), 1)
    oh_r = jnp.where(rc == pr % R2, 1.0, 0.0).astype(jnp.float32)
    cc = lax.broadcasted_iota(jnp.int32, (V3 * E3, NCOLS), 0)
    qq = lax.broadcasted_iota(jnp.int32, (V3 * E3, NCOLS), 1)
    hit = (cc == (qq // TROW) * E3 + qq % E3) & (((qq % TROW) // EMB) == 0)
    oh_c = jnp.where(hit, 1.0, 0.0).astype(jnp.float32)
    v = jnp.dot(jnp.dot(oh_r, c2f, preferred_element_type=jnp.float32),
                oh_c, preferred_element_type=jnp.float32)
    p2 = lax.broadcasted_iota(jnp.int32, (KDIM, NCOLS), 0)
    q2 = lax.broadcasted_iota(jnp.int32, (KDIM, NCOLS), 1)
    bd = jnp.where((p2 // R2) == ((q2 % EMB) // E3), v, 0.0)
    res = jnp.dot(w12_ref[...], bd, preferred_element_type=jnp.float32)
    out_ref[...] = res.reshape(OROWS, TROW)


def _build_table(core0, core1, core2):
    w12 = pl.pallas_call(
        _w12_body,
        out_shape=jax.ShapeDtypeStruct((NV12, KDIM), jnp.float32),
    )(core0, core1)
    c2f = core2.reshape(R2, V3 * E3)
    table = pl.pallas_call(
        _table_body,
        grid=(NV12 // ROWBLK,),
        in_specs=[
            pl.BlockSpec((ROWBLK, KDIM), lambda i: (i, 0)),
            pl.BlockSpec((R2, V3 * E3), lambda i: (0, 0)),
        ],
        out_specs=pl.BlockSpec((OROWS, TROW), lambda i: (i, 0)),
        out_shape=jax.ShapeDtypeStruct((VOCAB, TROW), jnp.float32),
    )(w12, c2f)
    return table


def _gather_rows(table, idx3):
    mesh = plsc.VectorSubcoreMesh(core_axis_name="c", subcore_axis_name="s")

    @functools.partial(
        pl.kernel,
        mesh=mesh,
        out_type=jax.ShapeDtypeStruct((B_TOTAL, TROW), jnp.float32),
        scratch_types=[
            pltpu.VMEM((NCH, KCH), jnp.int32),
            pltpu.VMEM((KCH, TROW), jnp.float32),
            pltpu.SemaphoreType.DMA,
        ],
    )
    def gather_k(table_hbm, idx_hbm, out_hbm, idx_v, rows_v, sem):
        wid = lax.axis_index("s") * NC + lax.axis_index("c")
        pltpu.sync_copy(idx_hbm.at[wid], idx_v)
        base = wid * B_PER_W

        def body(ch, carry):
            pltpu.async_copy(table_hbm.at[idx_v.at[ch]], rows_v, sem).wait()
            pltpu.sync_copy(rows_v, out_hbm.at[pl.ds(base + ch * KCH, KCH)])
            return carry

        lax.fori_loop(0, NCH, body, 0, unroll=False)

    return gather_k(table, idx3)


def kernel(core0, core1, core2, x):
    xshape = x.shape
    table = _build_table(core0, core1, core2)
    idx3 = x.reshape(NW, NCH, KCH).astype(jnp.int32)
    rows = _gather_rows(table, idx3)
    return rows[:, :EMB].reshape(xshape + (EMB,))


# speedup vs baseline: 1.6608x; 1.1219x over previous
import functools

import jax
import jax.numpy as jnp
from jax import lax
from jax.experimental import pallas as pl
from jax.experimental.pallas import tpu as pltpu
from jax.experimental.pallas import tpu_sc as plsc

V1, V2, V3 = 50, 40, 50
E1, E2, E3 = 4, 4, 4
R1, R2 = 16, 16
VOCAB = V1 * V2 * V3
EMB = E1 * E2 * E3
TROW = 2 * EMB
NV12 = V1 * V2
KDIM = E1 * E2 * R2
NCOLS = V3 * TROW
ROWBLK = 400
OROWS = ROWBLK * V3

NC, NS = 2, 16
NW = NC * NS
B_TOTAL = 4096 * 26
B_PER_W = B_TOTAL // NW
KCH = 128
NCH = B_PER_W // KCH


def _w12_body(c0_ref, c1_ref, out_ref):
    m0 = c0_ref[...].reshape(V1 * E1, R1)
    c1 = c1_ref[...].reshape(R1, V2 * E2 * R2)
    p = jnp.dot(m0, c1, preferred_element_type=jnp.float32)
    w12 = p.reshape(V1, E1, V2, E2 * R2).transpose(0, 2, 1, 3)
    out_ref[...] = w12.reshape(NV12, KDIM)


def _table_body(w12_ref, c2f_ref, out_ref, bd_ref):
    @pl.when(pl.program_id(0) == 0)
    def _():
        c2f = c2f_ref[...]
        pr = lax.broadcasted_iota(jnp.int32, (KDIM, R2), 0)
        rc = lax.broadcasted_iota(jnp.int32, (KDIM, R2), 1)
        oh_r = jnp.where(rc == pr % R2, 1.0, 0.0).astype(jnp.float32)
        cc = lax.broadcasted_iota(jnp.int32, (V3 * E3, NCOLS), 0)
        qq = lax.broadcasted_iota(jnp.int32, (V3 * E3, NCOLS), 1)
        hit = (cc == (qq // TROW) * E3 + qq % E3) & (((qq % TROW) // EMB) == 0)
        oh_c = jnp.where(hit, 1.0, 0.0).astype(jnp.float32)
        v = jnp.dot(jnp.dot(oh_r, c2f, preferred_element_type=jnp.float32),
                    oh_c, preferred_element_type=jnp.float32)
        p2 = lax.broadcasted_iota(jnp.int32, (KDIM, NCOLS), 0)
        q2 = lax.broadcasted_iota(jnp.int32, (KDIM, NCOLS), 1)
        bd_ref[...] = jnp.where((p2 // R2) == ((q2 % EMB) // E3), v, 0.0)

    res = jnp.dot(w12_ref[...], bd_ref[...], preferred_element_type=jnp.float32)
    out_ref[...] = res.reshape(OROWS, TROW)


def _build_table(core0, core1, core2):
    w12 = pl.pallas_call(
        _w12_body,
        out_shape=jax.ShapeDtypeStruct((NV12, KDIM), jnp.float32),
    )(core0, core1)
    c2f = core2.reshape(R2, V3 * E3)
    table = pl.pallas_call(
        _table_body,
        grid=(NV12 // ROWBLK,),
        in_specs=[
            pl.BlockSpec((ROWBLK, KDIM), lambda i: (i, 0)),
            pl.BlockSpec((R2, V3 * E3), lambda i: (0, 0)),
        ],
        out_specs=pl.BlockSpec((OROWS, TROW), lambda i: (i, 0)),
        out_shape=jax.ShapeDtypeStruct((VOCAB, TROW), jnp.float32),
        scratch_shapes=[pltpu.VMEM((KDIM, NCOLS), jnp.float32)],
    )(w12, c2f)
    return table


def _gather_rows(table, idx3):
    mesh = plsc.VectorSubcoreMesh(core_axis_name="c", subcore_axis_name="s")

    @functools.partial(
        pl.kernel,
        mesh=mesh,
        out_type=jax.ShapeDtypeStruct((B_TOTAL, TROW), jnp.float32),
        scratch_types=[
            pltpu.VMEM((NCH, KCH), jnp.int32),
            pltpu.VMEM((KCH, TROW), jnp.float32),
            pltpu.VMEM((KCH, TROW), jnp.float32),
            pltpu.SemaphoreType.DMA,
            pltpu.SemaphoreType.DMA,
        ],
    )
    def gather_k(table_hbm, idx_hbm, out_hbm, idx_v, rows_a, rows_b, sem_a, sem_b):
        wid = lax.axis_index("s") * NC + lax.axis_index("c")
        pltpu.sync_copy(idx_hbm.at[wid], idx_v)
        base = wid * B_PER_W
        bufs = (rows_a, rows_b)
        sems = (sem_a, sem_b)

        pltpu.make_async_copy(table_hbm.at[idx_v.at[0]], rows_a, sem_a).start()

        def body(ch0, carry):
            for b in range(2):
                ch = ch0 + b

                @pl.when(ch + 1 < NCH)
                def _():
                    pltpu.make_async_copy(
                        table_hbm.at[idx_v.at[ch + 1]], bufs[1 - b], sems[1 - b]
                    ).start()

                pltpu.make_async_copy(
                    table_hbm.at[idx_v.at[ch]], bufs[b], sems[b]
                ).wait()
                pltpu.sync_copy(bufs[b], out_hbm.at[pl.ds(base + ch * KCH, KCH)])
            return carry

        lax.fori_loop(0, NCH // 2, lambda i, c: body(i * 2, c), 0, unroll=False)

    return gather_k(table, idx3)


def kernel(core0, core1, core2, x):
    xshape = x.shape
    table = _build_table(core0, core1, core2)
    idx3 = x.reshape(NW, NCH, KCH).astype(jnp.int32)
    rows = _gather_rows(table, idx3)
    return rows[:, :EMB].reshape(xshape + (EMB,))
